# baseline (speedup 1.0000x reference)
import functools

import jax
import jax.numpy as jnp
from jax import lax
from jax.experimental import pallas as pl
from jax.experimental.pallas import tpu as pltpu
from jax.experimental.pallas import tpu_sc as plsc

K = 1024
D = 64
BETA = 0.25
B = 16
HW = 32 * 32
N_ROWS = B * HW

NC = 2
NS = 16
NW = NC * NS
IDX_CHUNK = 128


def _argmin_loss_body(x_ref, emb2_ref, a_ref, b_ref, inds_ref, loss_ref):
    x = x_ref[0]
    emb2 = emb2_ref[...]
    a = a_ref[0]
    b = b_ref[...]
    c2 = jax.lax.dot_general(emb2, x, (((1,), (0,)), ((), ())),
                             preferred_element_type=jnp.float32)
    dist = (a + b) - c2
    m = jnp.min(dist, axis=0, keepdims=True)
    iota_k = jax.lax.broadcasted_iota(jnp.int32, (K, HW), 0)
    sel = jnp.min(jnp.where(dist == m, iota_k, K), axis=0, keepdims=True)
    inds_ref[0] = sel

    @pl.when(pl.program_id(0) == 0)
    def _():
        loss_ref[...] = jnp.zeros_like(loss_ref)

    loss_ref[...] += jnp.sum(m, axis=(0, 1), keepdims=True)


def _argmin_loss(lat3, emb2, a3, b):
    nb = lat3.shape[0]
    return pl.pallas_call(
        _argmin_loss_body,
        grid=(nb,),
        in_specs=[
            pl.BlockSpec((1, D, HW), lambda i: (i, 0, 0)),
            pl.BlockSpec((K, D), lambda i: (0, 0)),
            pl.BlockSpec((1, 1, HW), lambda i: (i, 0, 0)),
            pl.BlockSpec((K, 1), lambda i: (0, 0)),
        ],
        out_specs=[
            pl.BlockSpec((1, 1, HW), lambda i: (i, 0, 0)),
            pl.BlockSpec((1, 1), lambda i: (0, 0)),
        ],
        out_shape=[
            jax.ShapeDtypeStruct((nb, 1, HW), jnp.int32),
            jax.ShapeDtypeStruct((1, 1), jnp.float32),
        ],
    )(lat3, emb2, a3, b)


def _make_sc_gather_body(rows_per_w, n_chunks):
    def body(table_hbm, idx_hbm, out_hbm, idx_v, rows_v, sem):
        wid = lax.axis_index("s") * NC + lax.axis_index("c")
        base = wid * rows_per_w
        pltpu.sync_copy(idx_hbm.at[wid], idx_v)
        copies = [
            pltpu.async_copy(table_hbm.at[idx_v.at[j]],
                             rows_v.at[pl.ds(j * IDX_CHUNK, IDX_CHUNK)],
                             sem)
            for j in range(n_chunks)
        ]
        for c in copies:
            c.wait()
        pltpu.sync_copy(rows_v, out_hbm.at[pl.ds(base, rows_per_w)])
    return body


def _sc_gather(embedding, idx):
    n = idx.shape[0]
    rows_per_w = n // NW
    n_chunks = rows_per_w // IDX_CHUNK
    mesh = plsc.VectorSubcoreMesh(core_axis_name="c", subcore_axis_name="s")
    f = functools.partial(
        pl.kernel,
        mesh=mesh,
        compiler_params=pltpu.CompilerParams(use_tc_tiling_on_sc=False),
        out_type=jax.ShapeDtypeStruct((n, D), jnp.float32),
        scratch_types=[
            pltpu.VMEM((n_chunks, IDX_CHUNK), jnp.int32),
            pltpu.VMEM((rows_per_w, D), jnp.float32),
            pltpu.SemaphoreType.DMA,
        ],
    )(_make_sc_gather_body(rows_per_w, n_chunks))
    return f(embedding, idx.reshape(NW, n_chunks, IDX_CHUNK))


def kernel(latents, embedding):
    lat = jnp.transpose(latents, (0, 2, 3, 1))
    flat = lat.reshape(-1, D)
    a = jnp.sum(flat ** 2, axis=1, keepdims=True)
    b = jnp.sum(embedding ** 2, axis=1)[:, None]
    lat3 = latents.reshape(B, D, HW)
    emb2 = embedding + embedding
    a3 = a.reshape(B, 1, HW)
    hb = B // 2
    inds0, loss0 = _argmin_loss(lat3[:hb], emb2, a3[:hb], b)
    inds1, loss1 = _argmin_loss(lat3[hb:], emb2, a3[hb:], b)
    q0 = _sc_gather(embedding, inds0.reshape(-1))
    q1 = _sc_gather(embedding, inds1.reshape(-1))
    loss_sum = loss0[0, 0] + loss1[0, 0]
    mean_sq = loss_sum / jnp.float32(N_ROWS * D)
    vq_loss = mean_sq * BETA + mean_sq
    t0 = jnp.transpose(q0.reshape(hb, 32, 32, D), (0, 3, 1, 2))
    t1 = jnp.transpose(q1.reshape(hb, 32, 32, D), (0, 3, 1, 2))
    out = jnp.concatenate([t0, t1], axis=0)
    return (out, vq_loss)
